# Initial kernel scaffold; baseline (speedup 1.0000x reference)
import functools

import jax
import jax.numpy as jnp
from jax import lax
from jax.experimental import pallas as pl
from jax.experimental.pallas import tpu as pltpu
from jax.experimental.pallas import tpu_sc as plsc

_VOCAB = 1000000
_D = 32
_B = 4096
_T = 200
_N = _B * _T
_NC = 2
_NS = 16
_NW = _NC * _NS
_PER_W = _N // _NW
_CHUNK = 1024
_NCHUNK = _PER_W // _CHUNK


@functools.partial(
    pl.kernel,
    out_type=jax.ShapeDtypeStruct((_N, _D), jnp.float32),
    mesh=plsc.VectorSubcoreMesh(core_axis_name="c", subcore_axis_name="s"),
    scratch_types=[
        pltpu.VMEM((_CHUNK,), jnp.int32),
        pltpu.VMEM((_CHUNK, _D), jnp.float32),
        pltpu.SemaphoreType.DMA,
    ],
)
def _gather_kernel(idx_hbm, table_hbm, out_hbm, idx_v, rows_v, sem):
    wid = lax.axis_index("s") * _NC + lax.axis_index("c")
    base = wid * _PER_W

    def body(g, carry):
        off = base + g * _CHUNK
        pltpu.sync_copy(idx_hbm.at[pl.ds(off, _CHUNK)], idx_v)
        pltpu.async_copy(table_hbm.at[idx_v], rows_v, sem).wait()
        pltpu.sync_copy(rows_v, out_hbm.at[pl.ds(off, _CHUNK)])
        return carry

    lax.fori_loop(0, _NCHUNK, body, 0)


def kernel(x, weight):
    flat_idx = x.reshape(_N)
    out = _gather_kernel(flat_idx, weight)
    return out.reshape(_B, _T, _D)


# speedup vs baseline: 1.4591x; 1.4591x over previous
import functools

import jax
import jax.numpy as jnp
from jax import lax
from jax.experimental import pallas as pl
from jax.experimental.pallas import tpu as pltpu
from jax.experimental.pallas import tpu_sc as plsc

_VOCAB = 1000000
_D = 32
_B = 4096
_T = 200
_N = _B * _T
_NC = 2
_NS = 16
_NW = _NC * _NS
_PER_W = _N // _NW
_CHUNK = 1024
_NCHUNK = _PER_W // _CHUNK


@functools.partial(
    pl.kernel,
    out_type=jax.ShapeDtypeStruct((_N, _D), jnp.float32),
    mesh=plsc.VectorSubcoreMesh(core_axis_name="c", subcore_axis_name="s"),
    scratch_types=[
        pltpu.VMEM((_CHUNK,), jnp.int32),
        pltpu.VMEM((_CHUNK, _D), jnp.float32),
        pltpu.SemaphoreType.DMA,
    ],
    compiler_params=pltpu.CompilerParams(use_tc_tiling_on_sc=False),
)
def _gather_kernel(idx_hbm, table_hbm, out_hbm, idx_v, rows_v, sem):
    wid = lax.axis_index("s") * _NC + lax.axis_index("c")
    base = wid * _PER_W

    def body(g, carry):
        off = base + g * _CHUNK
        pltpu.sync_copy(idx_hbm.at[pl.ds(off, _CHUNK)], idx_v)
        pltpu.async_copy(table_hbm.at[idx_v], rows_v, sem).wait()
        pltpu.sync_copy(rows_v, out_hbm.at[pl.ds(off, _CHUNK)])
        return carry

    lax.fori_loop(0, _NCHUNK, body, 0)


def kernel(x, weight):
    flat_idx = x.reshape(_N)
    out = _gather_kernel(flat_idx, weight)
    return out.reshape(_B, _T, _D)


# speedup vs baseline: 1.4862x; 1.0186x over previous
import functools

import jax
import jax.numpy as jnp
from jax import lax
from jax.experimental import pallas as pl
from jax.experimental.pallas import tpu as pltpu
from jax.experimental.pallas import tpu_sc as plsc

_VOCAB = 1000000
_D = 32
_B = 4096
_T = 200
_N = _B * _T
_NC = 2
_NS = 16
_NW = _NC * _NS
_PER_W = _N // _NW
_CHUNK = 800
_NBUF = 4
_NCHUNK = _PER_W // _CHUNK
_NOUT = _NCHUNK // _NBUF


@functools.partial(
    pl.kernel,
    out_type=jax.ShapeDtypeStruct((_N, _D), jnp.float32),
    mesh=plsc.VectorSubcoreMesh(core_axis_name="c", subcore_axis_name="s"),
    scratch_types=[
        pltpu.VMEM((_NBUF, _CHUNK), jnp.int32),
        pltpu.VMEM((_NBUF, _CHUNK, _D), jnp.float32),
    ] + [pltpu.SemaphoreType.DMA] * (2 * _NBUF),
    compiler_params=pltpu.CompilerParams(use_tc_tiling_on_sc=False),
)
def _gather_kernel(idx_hbm, table_hbm, out_hbm, idx_v, rows_v, *sems):
    gsem = sems[:_NBUF]
    ssem = sems[_NBUF:]
    wid = lax.axis_index("s") * _NC + lax.axis_index("c")
    base = wid * _PER_W

    for b in range(_NBUF):
        off = base + b * _CHUNK
        pltpu.sync_copy(idx_hbm.at[pl.ds(off, _CHUNK)], idx_v.at[b])
        pltpu.async_copy(table_hbm.at[idx_v.at[b]], rows_v.at[b], gsem[b])

    def body(j, carry):
        for b in range(_NBUF):
            g = j * _NBUF + b
            off = base + g * _CHUNK
            pltpu.make_async_copy(
                table_hbm.at[idx_v.at[b]], rows_v.at[b], gsem[b]
            ).wait()
            pltpu.async_copy(rows_v.at[b], out_hbm.at[pl.ds(off, _CHUNK)],
                             ssem[b])

            @pl.when(j < _NOUT - 1)
            def _():
                off2 = base + (g + _NBUF) * _CHUNK
                pltpu.make_async_copy(
                    rows_v.at[b], out_hbm.at[pl.ds(off, _CHUNK)], ssem[b]
                ).wait()
                pltpu.sync_copy(idx_hbm.at[pl.ds(off2, _CHUNK)], idx_v.at[b])
                pltpu.async_copy(table_hbm.at[idx_v.at[b]], rows_v.at[b],
                                 gsem[b])
        return carry

    lax.fori_loop(0, _NOUT, body, 0)

    for b in range(_NBUF):
        off = base + ((_NOUT - 1) * _NBUF + b) * _CHUNK
        pltpu.make_async_copy(
            rows_v.at[b], out_hbm.at[pl.ds(off, _CHUNK)], ssem[b]
        ).wait()


def kernel(x, weight):
    flat_idx = x.reshape(_N)
    out = _gather_kernel(flat_idx, weight)
    return out.reshape(_B, _T, _D)


# speedup vs baseline: 1.4864x; 1.0001x over previous
import functools

import jax
import jax.numpy as jnp
from jax import lax
from jax.experimental import pallas as pl
from jax.experimental.pallas import tpu as pltpu
from jax.experimental.pallas import tpu_sc as plsc

_VOCAB = 1000000
_D = 32
_B = 4096
_T = 200
_N = _B * _T
_NC = 2
_NS = 16
_NW = _NC * _NS
_PER_W = _B // _NW
_CB = 4
_NBUF = 4
_NCHUNK = _PER_W // _CB
_NOUT = _NCHUNK // _NBUF


@functools.partial(
    pl.kernel,
    out_type=jax.ShapeDtypeStruct((_B, _T, _D), jnp.float32),
    mesh=plsc.VectorSubcoreMesh(core_axis_name="c", subcore_axis_name="s"),
    scratch_types=[
        pltpu.VMEM((_NBUF, _CB, _T), jnp.int32),
        pltpu.VMEM((_NBUF, _CB, _T, _D), jnp.float32),
    ] + [pltpu.SemaphoreType.DMA] * (2 * _NBUF),
    compiler_params=pltpu.CompilerParams(use_tc_tiling_on_sc=False),
)
def _gather_kernel(idx_hbm, table_hbm, out_hbm, idx_v, rows_v, *sems):
    gsem = sems[:_NBUF]
    ssem = sems[_NBUF:]
    wid = lax.axis_index("s") * _NC + lax.axis_index("c")
    base = wid * _PER_W

    def start_gathers(b):
        for i in range(_CB):
            pltpu.async_copy(table_hbm.at[idx_v.at[b].at[i]],
                             rows_v.at[b].at[i], gsem[b])

    def wait_gathers(b):
        for i in range(_CB):
            pltpu.make_async_copy(table_hbm.at[idx_v.at[b].at[i]],
                                  rows_v.at[b].at[i], gsem[b]).wait()

    for b in range(_NBUF):
        off = base + b * _CB
        pltpu.sync_copy(idx_hbm.at[pl.ds(off, _CB)], idx_v.at[b])
        start_gathers(b)

    def body(j, carry):
        for b in range(_NBUF):
            g = j * _NBUF + b
            off = base + g * _CB
            wait_gathers(b)
            pltpu.async_copy(rows_v.at[b], out_hbm.at[pl.ds(off, _CB)],
                             ssem[b])

            @pl.when(j < _NOUT - 1)
            def _():
                off2 = base + (g + _NBUF) * _CB
                pltpu.make_async_copy(
                    rows_v.at[b], out_hbm.at[pl.ds(off, _CB)], ssem[b]
                ).wait()
                pltpu.sync_copy(idx_hbm.at[pl.ds(off2, _CB)], idx_v.at[b])
                start_gathers(b)
        return carry

    lax.fori_loop(0, _NOUT, body, 0)

    for b in range(_NBUF):
        off = base + ((_NOUT - 1) * _NBUF + b) * _CB
        pltpu.make_async_copy(
            rows_v.at[b], out_hbm.at[pl.ds(off, _CB)], ssem[b]
        ).wait()


def kernel(x, weight):
    return _gather_kernel(x, weight)
